# baseline (speedup 1.0000x reference)
import jax
import jax.numpy as jnp
from jax import lax
from jax.experimental import pallas as pl
from jax.experimental.pallas import tpu as pltpu
from jax.experimental.pallas import tpu_sc as plsc

NUM_CLASSES = 100000
FEAT = 64
BATCH = 16384
NC = 2
NS = 16
L = 16
NW = NC * NS
DIMS_PER_W = FEAT // NW
FCHUNK = 4096
NFC = BATCH // FCHUNK
UNROLL = 8
FRING = 3
NCHUNKS = DIMS_PER_W * NFC


def _body(featT_hbm, lab_hbm, centT_hbm, out_hbm,
          lab_v, tab_v, feat_v, shared, flat_v, pvec_v, out_v,
          sem_f0, sem_f1, sem_f2, sem_t, sem_l):
    c = lax.axis_index("c")
    s = lax.axis_index("s")
    wid = c * NS + s
    sem_f = (sem_f0, sem_f1, sem_f2)

    def fire_feat(g):
        d = wid * DIMS_PER_W + g // NFC
        return pltpu.async_copy(
            featT_hbm.at[d, pl.ds((g % NFC) * FCHUNK, FCHUNK)],
            feat_v.at[pl.ds((g % FRING) * FCHUNK, FCHUNK)],
            sem_f[g % FRING])

    tab_cp = pltpu.async_copy(centT_hbm.at[wid * DIMS_PER_W], tab_v, sem_t)
    feat_cps = {g: fire_feat(g) for g in range(FRING)}
    pltpu.async_copy(lab_hbm, lab_v, sem_l).wait()

    acc = jnp.zeros((L,), jnp.float32)
    for t in range(DIMS_PER_W):
        tab_cp.wait()
        for k in range(NFC):
            g = t * NFC + k
            feat_cps[g].wait()
            if g + FRING < NCHUNKS:
                feat_cps[g + FRING] = fire_feat(g + FRING)

            accs = (acc, jnp.zeros((L,), jnp.float32),
                    jnp.zeros((L,), jnp.float32),
                    jnp.zeros((L,), jnp.float32))

            @plsc.parallel_loop(0, FCHUNK // L, step=UNROLL, carry=accs)
            def accs(i, a, _k=k, _buf=g % FRING):
                a = list(a)
                for u in range(UNROLL):
                    idx = lab_v[pl.ds(_k * FCHUNK + (i + u) * L, L)]
                    cv = plsc.load_gather(tab_v, [idx])
                    fv = feat_v[pl.ds(_buf * FCHUNK + (i + u) * L, L)]
                    df = fv - cv
                    a[u % 4] = a[u % 4] + df * df
                return tuple(a)

            acc = accs[0] + accs[1] + accs[2] + accs[3]
        if t + 1 < DIMS_PER_W:
            tab_cp = pltpu.async_copy(centT_hbm.at[wid * DIMS_PER_W + t + 1],
                                      tab_v, sem_t)

    pvec_v[...] = acc
    pltpu.sync_copy(pvec_v, shared.at[pl.ds(s * L, L)])
    plsc.subcore_barrier()

    @pl.when(s == 0)
    def _():
        pltpu.sync_copy(shared, flat_v)

        def rstep(t, tot):
            return tot + flat_v[pl.ds(t * L, L)]

        total = lax.fori_loop(0, NS, rstep, jnp.zeros((L,), jnp.float32))
        out_v[...] = total * (1.0 / (BATCH * FEAT))
        pltpu.sync_copy(out_v, out_hbm.at[c])


@jax.jit
def _center_loss(features, labels, centers):
    featT = features.T
    centT = centers.T
    lab = labels.astype(jnp.int32)
    mesh = plsc.VectorSubcoreMesh(core_axis_name="c", subcore_axis_name="s")
    run = pl.kernel(
        _body,
        out_type=jax.ShapeDtypeStruct((NC, L), jnp.float32),
        mesh=mesh,
        scratch_types=[
            pltpu.VMEM((BATCH,), jnp.int32),
            pltpu.VMEM((NUM_CLASSES,), jnp.float32),
            pltpu.VMEM((FRING * FCHUNK,), jnp.float32),
            pltpu.VMEM_SHARED((NS * L,), jnp.float32),
            pltpu.VMEM((NS * L,), jnp.float32),
            pltpu.VMEM((L,), jnp.float32),
            pltpu.VMEM((L,), jnp.float32),
            pltpu.SemaphoreType.DMA,
            pltpu.SemaphoreType.DMA,
            pltpu.SemaphoreType.DMA,
            pltpu.SemaphoreType.DMA,
            pltpu.SemaphoreType.DMA,
        ],
        compiler_params=pltpu.CompilerParams(needs_layout_passes=False),
    )
    out = run(featT, lab, centT)
    return jnp.sum(out)


def kernel(features, labels, centers):
    return _center_loss(features, labels, centers)
